# Initial kernel scaffold; baseline (speedup 1.0000x reference)
import jax
import jax.numpy as jnp
from jax.experimental import pallas as pl


def kernel(z, embedding):
    raise NotImplementedError("write your pallas kernel here")


# speedup vs baseline: 1.0877x; 1.0877x over previous
import functools

import jax
import jax.numpy as jnp
from jax import lax
from jax.experimental import pallas as pl
from jax.experimental.pallas import tpu as pltpu
from jax.experimental.pallas import tpu_sc as plsc

K = 8192
D = 256
B = 8
HW = 1024
N = B * HW
KT = 512
NKT = K // KT
COMMITMENT_COST = 0.25


def _vq_argmin_body(z_ref, e_ref, code_ref, loss_ref,
                    zz_ref, min_ref, arg_ref, acc_ref):
    b = pl.program_id(0)
    k = pl.program_id(1)

    z_blk = z_ref[0]

    @pl.when(k == 0)
    def _():
        zz_ref[...] = jnp.sum(z_blk * z_blk, axis=0, keepdims=True)

    e_blk = e_ref[...]
    ee = jnp.sum(e_blk * e_blk, axis=1, keepdims=True)
    s = jnp.dot(e_blk, z_blk, preferred_element_type=jnp.float32)
    d = (zz_ref[...] - 2.0 * s) + ee

    dmin = jnp.min(d, axis=0, keepdims=True)
    gidx = lax.broadcasted_iota(jnp.int32, (KT, HW), 0) + k * KT
    cand = jnp.where(d == dmin, gidx, K)
    amin = jnp.min(cand, axis=0, keepdims=True)

    @pl.when(k == 0)
    def _():
        min_ref[...] = dmin
        arg_ref[...] = amin

    @pl.when(k > 0)
    def _():
        better = dmin < min_ref[...]
        arg_ref[...] = jnp.where(better, amin, arg_ref[...])
        min_ref[...] = jnp.minimum(dmin, min_ref[...])

    @pl.when(k == NKT - 1)
    def _():
        code_ref[...] = arg_ref[0]

        @pl.when(b == 0)
        def _():
            acc_ref[0, 0] = 0.0

        acc_ref[0, 0] += jnp.sum(min_ref[...])

        @pl.when(b == B - 1)
        def _():
            m = acc_ref[0, 0] / (N * D)
            loss_ref[0, 0] = m + m * COMMITMENT_COST


def _vq_argmin(z3, embedding):
    return pl.pallas_call(
        _vq_argmin_body,
        grid=(B, NKT),
        in_specs=[
            pl.BlockSpec((1, D, HW), lambda b, k: (b, 0, 0)),
            pl.BlockSpec((KT, D), lambda b, k: (k, 0)),
        ],
        out_specs=[
            pl.BlockSpec((HW,), lambda b, k: (b,)),
            pl.BlockSpec(memory_space=pltpu.SMEM),
        ],
        out_shape=[
            jax.ShapeDtypeStruct((N,), jnp.int32),
            jax.ShapeDtypeStruct((1, 1), jnp.float32),
        ],
        scratch_shapes=[
            pltpu.VMEM((1, HW), jnp.float32),
            pltpu.VMEM((1, HW), jnp.float32),
            pltpu.VMEM((1, HW), jnp.int32),
            pltpu.SMEM((1, 1), jnp.float32),
        ],
    )(z3, embedding)


def _make_sc_gather():
    info = plsc.get_sparse_core_info()
    nw = info.num_cores * info.num_subcores
    b_per_w = N // nw
    mesh = plsc.VectorSubcoreMesh(core_axis_name="c", subcore_axis_name="s")

    @functools.partial(
        pl.kernel, mesh=mesh,
        out_type=jax.ShapeDtypeStruct((N, D), jnp.float32),
        scratch_types=[
            pltpu.VMEM((b_per_w,), jnp.int32),
            pltpu.VMEM((b_per_w, D), jnp.float32),
            pltpu.SemaphoreType.DMA,
        ],
    )
    def gather(table_hbm, idx_hbm, out_hbm, idx_v, rows_v, sem):
        wid = lax.axis_index("s") * info.num_cores + lax.axis_index("c")
        base = wid * b_per_w
        pltpu.sync_copy(idx_hbm.at[pl.ds(base, b_per_w)], idx_v)
        pltpu.async_copy(table_hbm.at[idx_v], rows_v, sem).wait()
        pltpu.sync_copy(rows_v, out_hbm.at[pl.ds(base, b_per_w)])

    return gather


def kernel(z, embedding):
    z3 = z.reshape(B, D, HW)
    code_flat, loss = _vq_argmin(z3, embedding)
    zq_rows = _make_sc_gather()(embedding, code_flat)
    zq = zq_rows.reshape(B, 32, 32, D).transpose(0, 3, 1, 2)
    code = code_flat.reshape(B, 32, 32)
    return (zq, loss[0, 0], code)


# speedup vs baseline: 1.1087x; 1.0193x over previous
import functools

import jax
import jax.numpy as jnp
from jax import lax
from jax.experimental import pallas as pl
from jax.experimental.pallas import tpu as pltpu
from jax.experimental.pallas import tpu_sc as plsc

K = 8192
D = 256
B = 8
HW = 1024
N = B * HW
KT = 512
NKT = K // KT
COMMITMENT_COST = 0.25


def _vq_argmin_body(z_ref, e_ref, code_ref, loss_ref,
                    zz_ref, ee_ref, min_ref, arg_ref, acc_ref):
    b = pl.program_id(0)
    k = pl.program_id(1)

    z_blk = z_ref[0]

    @pl.when(k == 0)
    def _():
        zz_ref[...] = jnp.sum(z_blk * z_blk, axis=0, keepdims=True)

    e_blk = e_ref[...]

    @pl.when(b == 0)
    def _():
        ee_ref[k] = jnp.sum(e_blk * e_blk, axis=1, keepdims=True)

    s2 = jnp.dot(e_blk + e_blk, z_blk, preferred_element_type=jnp.float32)
    d = (zz_ref[...] - s2) + ee_ref[k]

    dmin = jnp.min(d, axis=0, keepdims=True)
    gidx = lax.broadcasted_iota(jnp.int32, (KT, HW), 0).astype(jnp.float32)
    cand = jnp.where(d == dmin, gidx, jnp.float32(KT))
    amin = jnp.min(cand, axis=0, keepdims=True) + jnp.float32(k * KT)

    @pl.when(k == 0)
    def _():
        min_ref[...] = dmin
        arg_ref[...] = amin

    @pl.when(k > 0)
    def _():
        better = dmin < min_ref[...]
        arg_ref[...] = jnp.where(better, amin, arg_ref[...])
        min_ref[...] = jnp.minimum(dmin, min_ref[...])

    @pl.when(k == NKT - 1)
    def _():
        code_ref[...] = arg_ref[0].astype(jnp.int32)

        @pl.when(b == 0)
        def _():
            acc_ref[0, 0] = 0.0

        acc_ref[0, 0] += jnp.sum(min_ref[...])

        @pl.when(b == B - 1)
        def _():
            m = acc_ref[0, 0] / (N * D)
            loss_ref[0, 0] = m + m * COMMITMENT_COST


def _vq_argmin(z3, embedding):
    return pl.pallas_call(
        _vq_argmin_body,
        grid=(B, NKT),
        in_specs=[
            pl.BlockSpec((1, D, HW), lambda b, k: (b, 0, 0)),
            pl.BlockSpec((KT, D), lambda b, k: (k, 0)),
        ],
        out_specs=[
            pl.BlockSpec((HW,), lambda b, k: (b,)),
            pl.BlockSpec(memory_space=pltpu.SMEM),
        ],
        out_shape=[
            jax.ShapeDtypeStruct((N,), jnp.int32),
            jax.ShapeDtypeStruct((1, 1), jnp.float32),
        ],
        scratch_shapes=[
            pltpu.VMEM((1, HW), jnp.float32),
            pltpu.VMEM((NKT, KT, 1), jnp.float32),
            pltpu.VMEM((1, HW), jnp.float32),
            pltpu.VMEM((1, HW), jnp.float32),
            pltpu.SMEM((1, 1), jnp.float32),
        ],
    )(z3, embedding)


def _make_sc_gather():
    info = plsc.get_sparse_core_info()
    nw = info.num_cores * info.num_subcores
    b_per_w = N // nw
    mesh = plsc.VectorSubcoreMesh(core_axis_name="c", subcore_axis_name="s")

    @functools.partial(
        pl.kernel, mesh=mesh,
        out_type=jax.ShapeDtypeStruct((N, D), jnp.float32),
        scratch_types=[
            pltpu.VMEM((b_per_w,), jnp.int32),
            pltpu.VMEM((b_per_w, D), jnp.float32),
            pltpu.SemaphoreType.DMA,
        ],
    )
    def gather(table_hbm, idx_hbm, out_hbm, idx_v, rows_v, sem):
        wid = lax.axis_index("s") * info.num_cores + lax.axis_index("c")
        base = wid * b_per_w
        pltpu.sync_copy(idx_hbm.at[pl.ds(base, b_per_w)], idx_v)
        pltpu.async_copy(table_hbm.at[idx_v], rows_v, sem).wait()
        pltpu.sync_copy(rows_v, out_hbm.at[pl.ds(base, b_per_w)])

    return gather


def kernel(z, embedding):
    z3 = z.reshape(B, D, HW)
    code_flat, loss = _vq_argmin(z3, embedding)
    zq_rows = _make_sc_gather()(embedding, code_flat)
    zq = zq_rows.reshape(B, 32, 32, D).transpose(0, 3, 1, 2)
    code = code_flat.reshape(B, 32, 32)
    return (zq, loss[0, 0], code)
